# Initial kernel scaffold; baseline (speedup 1.0000x reference)
import jax
import jax.numpy as jnp
from jax.experimental import pallas as pl


def kernel(x, edge_index, batch, msg_W0, msg_b0, msg_W1, msg_b1, msg_W2, msg_b2, gru_Wih0, gru_Whh0, gru_bih0, gru_bhh0, gru_Wih1, gru_Whh1, gru_bih1, gru_bhh1, gru_Wih2, gru_Whh2, gru_bih2, gru_bhh2, fc_W, fc_b):
    raise NotImplementedError("write your pallas kernel here")


# speedup vs baseline: 5.5823x; 5.5823x over previous
import functools

import jax
import jax.numpy as jnp
from jax import lax
from jax.experimental import pallas as pl
from jax.experimental.pallas import tpu as pltpu
from jax.experimental.pallas import tpu_sc as plsc

N = 10000
E = 320000
D = 128
H = 128
B = 64

NPAD = 10240
ROWS_PER_TILE = NPAD // 16
CHUNK = 80
NW = 32
EPW = E // NW
NCHUNK = EPW // CHUNK
BLK = 1000
GRID = N // BLK

_sc_mesh = plsc.VectorSubcoreMesh(core_axis_name="c", subcore_axis_name="s")


@functools.partial(
    pl.kernel,
    out_type=jax.ShapeDtypeStruct((2 * NPAD, H), jnp.float32),
    mesh=_sc_mesh,
    scratch_types=[
        pltpu.VMEM((CHUNK,), jnp.int32),
        pltpu.VMEM((CHUNK,), jnp.int32),
        pltpu.VMEM((CHUNK, H), jnp.float32),
        pltpu.VMEM((CHUNK, H), jnp.float32),
        pltpu.VMEM_SHARED((NPAD, H), jnp.float32),
        pltpu.SemaphoreType.DMA,
        pltpu.SemaphoreType.DMA,
    ],
)
def _sc_edge_agg(p_hbm, q_hbm, row_hbm, col_hbm, out_hbm,
                 rowi, coli, bufp, bufq, aggs, sem_p, sem_q):
    c = lax.axis_index("c")
    s = lax.axis_index("s")
    wid = s * 2 + c
    zbase = s * ROWS_PER_TILE

    zero = jnp.zeros((16,), jnp.float32)

    def _zrow(r, carry):
        for j in range(8):
            bufp[r, pl.ds(j * 16, 16)] = zero
        return carry

    lax.fori_loop(0, CHUNK, _zrow, 0)

    def _zcopy(k, carry):
        pltpu.sync_copy(bufp, aggs.at[pl.ds(zbase + k * CHUNK, CHUNK)])
        return carry

    lax.fori_loop(0, ROWS_PER_TILE // CHUNK, _zcopy, 0)
    plsc.subcore_barrier()

    ebase = wid * EPW

    def _chunk(i, carry):
        off = ebase + i * CHUNK
        pltpu.sync_copy(row_hbm.at[pl.ds(off, CHUNK)], rowi)
        pltpu.sync_copy(col_hbm.at[pl.ds(off, CHUNK)], coli)
        cp_p = pltpu.async_copy(p_hbm.at[rowi], bufp, sem_p)
        cp_q = pltpu.async_copy(q_hbm.at[coli], bufq, sem_q)
        cp_p.wait()
        cp_q.wait()

        def _rrow(r, cc):
            for j in range(8):
                v = bufp[r, pl.ds(j * 16, 16)] + bufq[r, pl.ds(j * 16, 16)]
                bufp[r, pl.ds(j * 16, 16)] = jnp.maximum(v, 0.0)
            return cc

        lax.fori_loop(0, CHUNK, _rrow, 0)
        pltpu.sync_copy(bufp, aggs.at[rowi], add=True)
        return carry

    lax.fori_loop(0, NCHUNK, _chunk, 0)
    plsc.subcore_barrier()

    pltpu.sync_copy(aggs.at[pl.ds(zbase, ROWS_PER_TILE)],
                    out_hbm.at[pl.ds(c * NPAD + zbase, ROWS_PER_TILE)])


def _stage_a_body(x_ref, wt_ref, wb_ref, b_ref, p_ref, q_ref):
    xb = x_ref[...]
    p_ref[...] = jnp.dot(xb, wt_ref[...], preferred_element_type=jnp.float32)
    q_ref[...] = (jnp.dot(xb, wb_ref[...], preferred_element_type=jnp.float32)
                  + b_ref[...])


_stage_a = pl.pallas_call(
    _stage_a_body,
    grid=(GRID,),
    in_specs=[
        pl.BlockSpec((BLK, D), lambda i: (i, 0)),
        pl.BlockSpec((D, H), lambda i: (0, 0)),
        pl.BlockSpec((D, H), lambda i: (0, 0)),
        pl.BlockSpec((1, H), lambda i: (0, 0)),
    ],
    out_specs=[
        pl.BlockSpec((BLK, H), lambda i: (i, 0)),
        pl.BlockSpec((BLK, H), lambda i: (i, 0)),
    ],
    out_shape=[
        jax.ShapeDtypeStruct((N, H), jnp.float32),
        jax.ShapeDtypeStruct((N, H), jnp.float32),
    ],
)


def _gru(agg, xb, wih, whh, bih, bhh):
    gi = lax.dot_general(agg, wih, (((1,), (1,)), ((), ())),
                         preferred_element_type=jnp.float32) + bih
    gh = lax.dot_general(xb, whh, (((1,), (1,)), ((), ())),
                         preferred_element_type=jnp.float32) + bhh
    r = jax.nn.sigmoid(gi[:, :H] + gh[:, :H])
    z = jax.nn.sigmoid(gi[:, H:2 * H] + gh[:, H:2 * H])
    n = jnp.tanh(gi[:, 2 * H:] + r * gh[:, 2 * H:])
    return (1.0 - z) * n + z * xb


def _stage_ba_body(agg_ref, x_ref, wih_ref, whh_ref, bih_ref, bhh_ref,
                   wtn_ref, wbn_ref, bn_ref, xn_ref, pn_ref, qn_ref):
    agg = agg_ref[0] + agg_ref[1]
    xn = _gru(agg, x_ref[...], wih_ref[...], whh_ref[...],
              bih_ref[...], bhh_ref[...])
    xn_ref[...] = xn
    pn_ref[...] = jnp.dot(xn, wtn_ref[...], preferred_element_type=jnp.float32)
    qn_ref[...] = (jnp.dot(xn, wbn_ref[...], preferred_element_type=jnp.float32)
                   + bn_ref[...])


_stage_ba = pl.pallas_call(
    _stage_ba_body,
    grid=(GRID,),
    in_specs=[
        pl.BlockSpec((2, BLK, H), lambda i: (0, i, 0)),
        pl.BlockSpec((BLK, H), lambda i: (i, 0)),
        pl.BlockSpec((3 * H, H), lambda i: (0, 0)),
        pl.BlockSpec((3 * H, H), lambda i: (0, 0)),
        pl.BlockSpec((1, 3 * H), lambda i: (0, 0)),
        pl.BlockSpec((1, 3 * H), lambda i: (0, 0)),
        pl.BlockSpec((D, H), lambda i: (0, 0)),
        pl.BlockSpec((D, H), lambda i: (0, 0)),
        pl.BlockSpec((1, H), lambda i: (0, 0)),
    ],
    out_specs=[
        pl.BlockSpec((BLK, H), lambda i: (i, 0)),
        pl.BlockSpec((BLK, H), lambda i: (i, 0)),
        pl.BlockSpec((BLK, H), lambda i: (i, 0)),
    ],
    out_shape=[
        jax.ShapeDtypeStruct((N, H), jnp.float32),
        jax.ShapeDtypeStruct((N, H), jnp.float32),
        jax.ShapeDtypeStruct((N, H), jnp.float32),
    ],
)


def _final_body(agg_ref, x_ref, wih_ref, whh_ref, bih_ref, bhh_ref,
                batch_ref, fcw_ref, fcb_ref, out_ref, pool_acc, cnt_acc):
    i = pl.program_id(0)
    agg = agg_ref[0] + agg_ref[1]
    xn = _gru(agg, x_ref[...], wih_ref[...], whh_ref[...],
              bih_ref[...], bhh_ref[...])
    seg = batch_ref[...]
    lanes = lax.broadcasted_iota(jnp.int32, (BLK, B), 1).astype(jnp.float32)
    onehot = jnp.where(seg == lanes, 1.0, 0.0)
    pooled = lax.dot_general(onehot, xn, (((0,), (0,)), ((), ())),
                             preferred_element_type=jnp.float32)
    ones = jnp.ones((BLK, 1), dtype=jnp.float32)
    cnt = lax.dot_general(onehot, ones, (((0,), (0,)), ((), ())),
                          preferred_element_type=jnp.float32)

    @pl.when(i == 0)
    def _():
        pool_acc[...] = jnp.zeros_like(pool_acc)
        cnt_acc[...] = jnp.zeros_like(cnt_acc)

    pool_acc[...] += pooled
    cnt_acc[...] += cnt

    @pl.when(i == GRID - 1)
    def _():
        mean = pool_acc[...] / jnp.maximum(cnt_acc[...], 1.0)
        out_ref[...] = (jnp.dot(mean, fcw_ref[...],
                                preferred_element_type=jnp.float32)
                        + fcb_ref[...])


_final = pl.pallas_call(
    _final_body,
    grid=(GRID,),
    in_specs=[
        pl.BlockSpec((2, BLK, H), lambda i: (0, i, 0)),
        pl.BlockSpec((BLK, H), lambda i: (i, 0)),
        pl.BlockSpec((3 * H, H), lambda i: (0, 0)),
        pl.BlockSpec((3 * H, H), lambda i: (0, 0)),
        pl.BlockSpec((1, 3 * H), lambda i: (0, 0)),
        pl.BlockSpec((1, 3 * H), lambda i: (0, 0)),
        pl.BlockSpec((BLK, 1), lambda i: (i, 0)),
        pl.BlockSpec((H, 1), lambda i: (0, 0)),
        pl.BlockSpec((1, 1), lambda i: (0, 0)),
    ],
    out_specs=pl.BlockSpec((B, 1), lambda i: (0, 0)),
    out_shape=jax.ShapeDtypeStruct((B, 1), jnp.float32),
    scratch_shapes=[
        pltpu.VMEM((B, H), jnp.float32),
        pltpu.VMEM((B, 1), jnp.float32),
    ],
)


def kernel(x, edge_index, batch, msg_W0, msg_b0, msg_W1, msg_b1, msg_W2,
           msg_b2, gru_Wih0, gru_Whh0, gru_bih0, gru_bhh0, gru_Wih1, gru_Whh1,
           gru_bih1, gru_bhh1, gru_Wih2, gru_Whh2, gru_bih2, gru_bhh2,
           fc_W, fc_b):
    row = edge_index[0]
    col = edge_index[1]
    batch_f = batch.astype(jnp.float32).reshape(N, 1)

    p, q = _stage_a(x, msg_W0[:D], msg_W0[D:], msg_b0.reshape(1, H))
    agg = _sc_edge_agg(p, q, row, col).reshape(2, NPAD, H)
    x1, p1, q1 = _stage_ba(agg, x, gru_Wih0, gru_Whh0,
                           gru_bih0.reshape(1, 3 * H), gru_bhh0.reshape(1, 3 * H),
                           msg_W1[:H], msg_W1[H:], msg_b1.reshape(1, H))
    agg = _sc_edge_agg(p1, q1, row, col).reshape(2, NPAD, H)
    x2, p2, q2 = _stage_ba(agg, x1, gru_Wih1, gru_Whh1,
                           gru_bih1.reshape(1, 3 * H), gru_bhh1.reshape(1, 3 * H),
                           msg_W2[:H], msg_W2[H:], msg_b2.reshape(1, H))
    agg = _sc_edge_agg(p2, q2, row, col).reshape(2, NPAD, H)
    out = _final(agg, x2, gru_Wih2, gru_Whh2,
                 gru_bih2.reshape(1, 3 * H), gru_bhh2.reshape(1, 3 * H),
                 batch_f, fc_W, fc_b.reshape(1, 1))
    return out.reshape(-1)


# speedup vs baseline: 10.2397x; 1.8343x over previous
import functools

import jax
import jax.numpy as jnp
from jax import lax
from jax.experimental import pallas as pl
from jax.experimental.pallas import tpu as pltpu
from jax.experimental.pallas import tpu_sc as plsc

N = 10000
E = 320000
D = 128
H = 128
B = 64

NPAD = 10240
ROWS_PER_TILE = NPAD // 16
CHUNK = 80
NW = 32
EPW = E // NW
NCHUNK = EPW // CHUNK
NOUTER = (NCHUNK + 1) // 2
BLK = 1000
GRID = N // BLK

_sc_mesh = plsc.VectorSubcoreMesh(core_axis_name="c", subcore_axis_name="s")


@functools.partial(
    pl.kernel,
    out_type=jax.ShapeDtypeStruct((2 * NPAD, H), jnp.float32),
    mesh=_sc_mesh,
    scratch_types=[
        pltpu.VMEM((2, CHUNK), jnp.int32),
        pltpu.VMEM((2, CHUNK), jnp.int32),
        pltpu.VMEM((CHUNK, H), jnp.float32),
        pltpu.VMEM((CHUNK, H), jnp.float32),
        pltpu.VMEM((CHUNK, H), jnp.float32),
        pltpu.VMEM((CHUNK, H), jnp.float32),
        pltpu.VMEM_SHARED((NPAD, H), jnp.float32),
        pltpu.SemaphoreType.DMA,
        pltpu.SemaphoreType.DMA,
        pltpu.SemaphoreType.DMA,
        pltpu.SemaphoreType.DMA,
        pltpu.SemaphoreType.DMA,
        pltpu.SemaphoreType.DMA,
    ],
)
def _sc_edge_agg(p_hbm, q_hbm, rc_hbm, out_hbm,
                 rci0, rci1, bufp0, bufq0, bufp1, bufq1, aggs,
                 semp0, semq0, semp1, semq1, sems0, sems1):
    c = lax.axis_index("c")
    s = lax.axis_index("s")
    wid = s * 2 + c
    zbase = s * ROWS_PER_TILE

    zero = jnp.zeros((16,), jnp.float32)

    def _zrow(r, carry):
        for j in range(8):
            bufp0[r, pl.ds(j * 16, 16)] = zero
        return carry

    lax.fori_loop(0, CHUNK, _zrow, 0)
    nfull = ROWS_PER_TILE // CHUNK
    for k in range(nfull):
        pltpu.sync_copy(bufp0, aggs.at[pl.ds(zbase + k * CHUNK, CHUNK)])
    rem = ROWS_PER_TILE - nfull * CHUNK
    if rem:
        pltpu.sync_copy(bufp0.at[pl.ds(0, rem)],
                        aggs.at[pl.ds(zbase + nfull * CHUNK, rem)])
    plsc.subcore_barrier()

    rcis = (rci0, rci1)
    bufps = (bufp0, bufp1)
    bufqs = (bufq0, bufq1)
    semps = (semp0, semp1)
    semqs = (semq0, semq1)
    semss = (sems0, sems1)

    def _fetch(ci, b):
        pltpu.sync_copy(rc_hbm.at[wid, ci], rcis[b])
        pltpu.async_copy(p_hbm.at[rcis[b].at[0]], bufps[b], semps[b])
        pltpu.async_copy(q_hbm.at[rcis[b].at[1]], bufqs[b], semqs[b])

    _fetch(0, 0)

    def _outer(i, carry):
        for b in range(2):
            ci = i * 2 + b
            nb = (b + 1) % 2

            @pl.when(ci + 1 < NCHUNK)
            def _():
                @pl.when(ci >= 1)
                def _():
                    pltpu.make_async_copy(
                        bufps[nb], aggs.at[rcis[nb].at[0]], semss[nb]).wait()
                _fetch(ci + 1, nb)

            @pl.when(ci < NCHUNK)
            def _():
                pltpu.make_async_copy(p_hbm.at[rcis[b].at[0]],
                                      bufps[b], semps[b]).wait()
                pltpu.make_async_copy(q_hbm.at[rcis[b].at[1]],
                                      bufqs[b], semqs[b]).wait()

                def _rrow(r, cc):
                    for j in range(8):
                        v = (bufps[b][r, pl.ds(j * 16, 16)]
                             + bufqs[b][r, pl.ds(j * 16, 16)])
                        bufps[b][r, pl.ds(j * 16, 16)] = jnp.maximum(v, 0.0)
                    return cc

                lax.fori_loop(0, CHUNK, _rrow, 0)
                pltpu.async_copy(bufps[b], aggs.at[rcis[b].at[0]],
                                 semss[b], add=True)
        return carry

    lax.fori_loop(0, NOUTER, _outer, 0)
    pltpu.make_async_copy(bufps[0], aggs.at[rcis[0].at[0]], semss[0]).wait()
    pltpu.make_async_copy(bufps[1], aggs.at[rcis[1].at[0]], semss[1]).wait()
    plsc.subcore_barrier()

    pltpu.sync_copy(aggs.at[pl.ds(zbase, ROWS_PER_TILE)],
                    out_hbm.at[pl.ds(c * NPAD + zbase, ROWS_PER_TILE)])


def _stage_a_body(x_ref, wt_ref, wb_ref, b_ref, p_ref, q_ref):
    xb = x_ref[...]
    p_ref[...] = jnp.dot(xb, wt_ref[...], preferred_element_type=jnp.float32)
    q_ref[...] = (jnp.dot(xb, wb_ref[...], preferred_element_type=jnp.float32)
                  + b_ref[...])


_stage_a = pl.pallas_call(
    _stage_a_body,
    grid=(GRID,),
    in_specs=[
        pl.BlockSpec((BLK, D), lambda i: (i, 0)),
        pl.BlockSpec((D, H), lambda i: (0, 0)),
        pl.BlockSpec((D, H), lambda i: (0, 0)),
        pl.BlockSpec((1, H), lambda i: (0, 0)),
    ],
    out_specs=[
        pl.BlockSpec((BLK, H), lambda i: (i, 0)),
        pl.BlockSpec((BLK, H), lambda i: (i, 0)),
    ],
    out_shape=[
        jax.ShapeDtypeStruct((N, H), jnp.float32),
        jax.ShapeDtypeStruct((N, H), jnp.float32),
    ],
)


def _gru(agg, xb, wih, whh, bih, bhh):
    gi = lax.dot_general(agg, wih, (((1,), (1,)), ((), ())),
                         preferred_element_type=jnp.float32) + bih
    gh = lax.dot_general(xb, whh, (((1,), (1,)), ((), ())),
                         preferred_element_type=jnp.float32) + bhh
    r = jax.nn.sigmoid(gi[:, :H] + gh[:, :H])
    z = jax.nn.sigmoid(gi[:, H:2 * H] + gh[:, H:2 * H])
    n = jnp.tanh(gi[:, 2 * H:] + r * gh[:, 2 * H:])
    return (1.0 - z) * n + z * xb


def _stage_ba_body(agg_ref, x_ref, wih_ref, whh_ref, bih_ref, bhh_ref,
                   wtn_ref, wbn_ref, bn_ref, xn_ref, pn_ref, qn_ref):
    agg = agg_ref[0] + agg_ref[1]
    xn = _gru(agg, x_ref[...], wih_ref[...], whh_ref[...],
              bih_ref[...], bhh_ref[...])
    xn_ref[...] = xn
    pn_ref[...] = jnp.dot(xn, wtn_ref[...], preferred_element_type=jnp.float32)
    qn_ref[...] = (jnp.dot(xn, wbn_ref[...], preferred_element_type=jnp.float32)
                   + bn_ref[...])


_stage_ba = pl.pallas_call(
    _stage_ba_body,
    grid=(GRID,),
    in_specs=[
        pl.BlockSpec((2, BLK, H), lambda i: (0, i, 0)),
        pl.BlockSpec((BLK, H), lambda i: (i, 0)),
        pl.BlockSpec((3 * H, H), lambda i: (0, 0)),
        pl.BlockSpec((3 * H, H), lambda i: (0, 0)),
        pl.BlockSpec((1, 3 * H), lambda i: (0, 0)),
        pl.BlockSpec((1, 3 * H), lambda i: (0, 0)),
        pl.BlockSpec((D, H), lambda i: (0, 0)),
        pl.BlockSpec((D, H), lambda i: (0, 0)),
        pl.BlockSpec((1, H), lambda i: (0, 0)),
    ],
    out_specs=[
        pl.BlockSpec((BLK, H), lambda i: (i, 0)),
        pl.BlockSpec((BLK, H), lambda i: (i, 0)),
        pl.BlockSpec((BLK, H), lambda i: (i, 0)),
    ],
    out_shape=[
        jax.ShapeDtypeStruct((N, H), jnp.float32),
        jax.ShapeDtypeStruct((N, H), jnp.float32),
        jax.ShapeDtypeStruct((N, H), jnp.float32),
    ],
)


def _final_body(agg_ref, x_ref, wih_ref, whh_ref, bih_ref, bhh_ref,
                batch_ref, fcw_ref, fcb_ref, out_ref, pool_acc, cnt_acc):
    i = pl.program_id(0)
    agg = agg_ref[0] + agg_ref[1]
    xn = _gru(agg, x_ref[...], wih_ref[...], whh_ref[...],
              bih_ref[...], bhh_ref[...])
    seg = batch_ref[...]
    lanes = lax.broadcasted_iota(jnp.int32, (BLK, B), 1).astype(jnp.float32)
    onehot = jnp.where(seg == lanes, 1.0, 0.0)
    pooled = lax.dot_general(onehot, xn, (((0,), (0,)), ((), ())),
                             preferred_element_type=jnp.float32)
    ones = jnp.ones((BLK, 1), dtype=jnp.float32)
    cnt = lax.dot_general(onehot, ones, (((0,), (0,)), ((), ())),
                          preferred_element_type=jnp.float32)

    @pl.when(i == 0)
    def _():
        pool_acc[...] = jnp.zeros_like(pool_acc)
        cnt_acc[...] = jnp.zeros_like(cnt_acc)

    pool_acc[...] += pooled
    cnt_acc[...] += cnt

    @pl.when(i == GRID - 1)
    def _():
        mean = pool_acc[...] / jnp.maximum(cnt_acc[...], 1.0)
        out_ref[...] = (jnp.dot(mean, fcw_ref[...],
                                preferred_element_type=jnp.float32)
                        + fcb_ref[...])


_final = pl.pallas_call(
    _final_body,
    grid=(GRID,),
    in_specs=[
        pl.BlockSpec((2, BLK, H), lambda i: (0, i, 0)),
        pl.BlockSpec((BLK, H), lambda i: (i, 0)),
        pl.BlockSpec((3 * H, H), lambda i: (0, 0)),
        pl.BlockSpec((3 * H, H), lambda i: (0, 0)),
        pl.BlockSpec((1, 3 * H), lambda i: (0, 0)),
        pl.BlockSpec((1, 3 * H), lambda i: (0, 0)),
        pl.BlockSpec((BLK, 1), lambda i: (i, 0)),
        pl.BlockSpec((H, 1), lambda i: (0, 0)),
        pl.BlockSpec((1, 1), lambda i: (0, 0)),
    ],
    out_specs=pl.BlockSpec((B, 1), lambda i: (0, 0)),
    out_shape=jax.ShapeDtypeStruct((B, 1), jnp.float32),
    scratch_shapes=[
        pltpu.VMEM((B, H), jnp.float32),
        pltpu.VMEM((B, 1), jnp.float32),
    ],
)


def kernel(x, edge_index, batch, msg_W0, msg_b0, msg_W1, msg_b1, msg_W2,
           msg_b2, gru_Wih0, gru_Whh0, gru_bih0, gru_bhh0, gru_Wih1, gru_Whh1,
           gru_bih1, gru_bhh1, gru_Wih2, gru_Whh2, gru_bih2, gru_bhh2,
           fc_W, fc_b):
    rc = edge_index.reshape(2, NW, NCHUNK, CHUNK).transpose(1, 2, 0, 3)
    batch_f = batch.astype(jnp.float32).reshape(N, 1)

    p, q = _stage_a(x, msg_W0[:D], msg_W0[D:], msg_b0.reshape(1, H))
    agg = _sc_edge_agg(p, q, rc).reshape(2, NPAD, H)
    x1, p1, q1 = _stage_ba(agg, x, gru_Wih0, gru_Whh0,
                           gru_bih0.reshape(1, 3 * H), gru_bhh0.reshape(1, 3 * H),
                           msg_W1[:H], msg_W1[H:], msg_b1.reshape(1, H))
    agg = _sc_edge_agg(p1, q1, rc).reshape(2, NPAD, H)
    x2, p2, q2 = _stage_ba(agg, x1, gru_Wih1, gru_Whh1,
                           gru_bih1.reshape(1, 3 * H), gru_bhh1.reshape(1, 3 * H),
                           msg_W2[:H], msg_W2[H:], msg_b2.reshape(1, H))
    agg = _sc_edge_agg(p2, q2, rc).reshape(2, NPAD, H)
    out = _final(agg, x2, gru_Wih2, gru_Whh2,
                 gru_bih2.reshape(1, 3 * H), gru_bhh2.reshape(1, 3 * H),
                 batch_f, fc_W, fc_b.reshape(1, 1))
    return out.reshape(-1)
